# baseline (speedup 1.0000x reference)
import functools

import jax
import jax.numpy as jnp
from jax.experimental import pallas as pl

_DIM = 256
_NE = 1024
_BLK = 512


def _vq_kernel(x_ref, e_ref, q_ref, idx_ref, diff_ref):
    x = x_ref[...]
    e = e_ref[...]
    xsq = jnp.sum(x * x, axis=1, keepdims=True)
    esq = jnp.sum(e * e, axis=0, keepdims=True)
    xe = jnp.dot(x, e, preferred_element_type=jnp.float32)
    dist = xsq - 2.0 * xe + esq
    minv = jnp.min(dist, axis=1, keepdims=True)
    lane_f = jax.lax.broadcasted_iota(
        jnp.int32, (_BLK, _NE), 1).astype(jnp.float32)
    idx_f = jnp.min(jnp.where(dist == minv, lane_f, jnp.float32(_NE)),
                    axis=1, keepdims=True)
    idx_ref[...] = idx_f.astype(jnp.int32)

    onehot = (lane_f == idx_f).astype(jnp.bfloat16)
    e_hi = e.astype(jnp.bfloat16)
    e_lo = (e - e_hi.astype(jnp.float32)).astype(jnp.bfloat16)
    dims = (((1,), (1,)), ((), ()))
    q = (jax.lax.dot_general(onehot, e_hi, dims,
                             preferred_element_type=jnp.float32)
         + jax.lax.dot_general(onehot, e_lo, dims,
                               preferred_element_type=jnp.float32))
    q_ref[...] = x + (q - x)

    d = jnp.sum(minv).reshape(1, 1)

    @pl.when(pl.program_id(0) == 0)
    def _():
        diff_ref[...] = jnp.zeros((1, 1), jnp.float32)

    diff_ref[...] += d


def kernel(input, embed):
    flat = input.reshape(-1, _DIM)
    n_tok = flat.shape[0]
    nblk = n_tok // _BLK
    q, idx3, diff = pl.pallas_call(
        _vq_kernel,
        grid=(nblk,),
        in_specs=[
            pl.BlockSpec((_BLK, _DIM), lambda i: (i, 0)),
            pl.BlockSpec((_DIM, _NE), lambda i: (0, 0)),
        ],
        out_specs=[
            pl.BlockSpec((_BLK, _DIM), lambda i: (i, 0)),
            pl.BlockSpec((_BLK, 1), lambda i: (i, 0)),
            pl.BlockSpec((1, 1), lambda i: (0, 0)),
        ],
        out_shape=[
            jax.ShapeDtypeStruct((n_tok, _DIM), jnp.float32),
            jax.ShapeDtypeStruct((n_tok, 1), jnp.int32),
            jax.ShapeDtypeStruct((1, 1), jnp.float32),
        ],
    )(flat, embed)
    quantize = q.reshape(input.shape)
    embed_ind = idx3.reshape(input.shape[:-1])
    diff_scalar = diff[0, 0] / jnp.float32(n_tok * _DIM)
    return (quantize, diff_scalar, embed_ind)


# speedup vs baseline: 1.1258x; 1.1258x over previous
import functools

import jax
import jax.numpy as jnp
from jax.experimental import pallas as pl

_DIM = 256
_NE = 1024
_BLK = 1024


def _vq_kernel(x_ref, e_ref, q_ref, idx_ref, diff_ref):
    x = x_ref[...]
    e = e_ref[...]
    xsq = jnp.sum(x * x, axis=1, keepdims=True)
    esq = jnp.sum(e * e, axis=0, keepdims=True)
    xe = jnp.dot(x, e, preferred_element_type=jnp.float32)
    dist = xsq - 2.0 * xe + esq
    minv = jnp.min(dist, axis=1, keepdims=True)
    lane_f = jax.lax.broadcasted_iota(
        jnp.int32, (_BLK, _NE), 1).astype(jnp.float32)
    idx_f = jnp.min(jnp.where(dist == minv, lane_f, jnp.float32(_NE)),
                    axis=1, keepdims=True)
    idx_ref[...] = idx_f.astype(jnp.int32)

    onehot = (lane_f == idx_f).astype(jnp.bfloat16)
    e_hi = e.astype(jnp.bfloat16)
    e_lo = (e - e_hi.astype(jnp.float32)).astype(jnp.bfloat16)
    dims = (((1,), (1,)), ((), ()))
    q = (jax.lax.dot_general(onehot, e_hi, dims,
                             preferred_element_type=jnp.float32)
         + jax.lax.dot_general(onehot, e_lo, dims,
                               preferred_element_type=jnp.float32))
    q_ref[...] = q

    d = jnp.sum(minv).reshape(1, 1)

    @pl.when(pl.program_id(0) == 0)
    def _():
        diff_ref[...] = jnp.zeros((1, 1), jnp.float32)

    diff_ref[...] += d


def kernel(input, embed):
    flat = input.reshape(-1, _DIM)
    n_tok = flat.shape[0]
    nblk = n_tok // _BLK
    q, idx3, diff = pl.pallas_call(
        _vq_kernel,
        grid=(nblk,),
        in_specs=[
            pl.BlockSpec((_BLK, _DIM), lambda i: (i, 0)),
            pl.BlockSpec((_DIM, _NE), lambda i: (0, 0)),
        ],
        out_specs=[
            pl.BlockSpec((_BLK, _DIM), lambda i: (i, 0)),
            pl.BlockSpec((_BLK, 1), lambda i: (i, 0)),
            pl.BlockSpec((1, 1), lambda i: (0, 0)),
        ],
        out_shape=[
            jax.ShapeDtypeStruct((n_tok, _DIM), jnp.float32),
            jax.ShapeDtypeStruct((n_tok, 1), jnp.int32),
            jax.ShapeDtypeStruct((1, 1), jnp.float32),
        ],
    )(flat, embed)
    quantize = q.reshape(input.shape)
    embed_ind = idx3.reshape(input.shape[:-1])
    diff_scalar = diff[0, 0] / jnp.float32(n_tok * _DIM)
    return (quantize, diff_scalar, embed_ind)


# speedup vs baseline: 1.1731x; 1.0420x over previous
import functools

import jax
import jax.numpy as jnp
from jax.experimental import pallas as pl

_DIM = 256
_NE = 1024
_BLK = 2048


def _vq_kernel(x_ref, e_ref, q_ref, idx_ref, diff_ref):
    x = x_ref[...]
    e = e_ref[...]
    xsq = jnp.sum(x * x, axis=1, keepdims=True)
    esq = jnp.sum(e * e, axis=0, keepdims=True)
    xe = jnp.dot(x, e, preferred_element_type=jnp.float32)
    dist = xsq - 2.0 * xe + esq
    minv = jnp.min(dist, axis=1, keepdims=True)
    lane_f = jax.lax.broadcasted_iota(
        jnp.int32, (_BLK, _NE), 1).astype(jnp.float32)
    idx_f = jnp.min(jnp.where(dist == minv, lane_f, jnp.float32(_NE)),
                    axis=1, keepdims=True)
    idx_ref[...] = idx_f.astype(jnp.int32)

    onehot = (lane_f == idx_f).astype(jnp.bfloat16)
    e_hi = e.astype(jnp.bfloat16)
    e_lo = (e - e_hi.astype(jnp.float32)).astype(jnp.bfloat16)
    dims = (((1,), (1,)), ((), ()))
    q = (jax.lax.dot_general(onehot, e_hi, dims,
                             preferred_element_type=jnp.float32)
         + jax.lax.dot_general(onehot, e_lo, dims,
                               preferred_element_type=jnp.float32))
    q_ref[...] = q

    d = jnp.sum(minv).reshape(1, 1)

    @pl.when(pl.program_id(0) == 0)
    def _():
        diff_ref[...] = jnp.zeros((1, 1), jnp.float32)

    diff_ref[...] += d


def kernel(input, embed):
    flat = input.reshape(-1, _DIM)
    n_tok = flat.shape[0]
    nblk = n_tok // _BLK
    q, idx3, diff = pl.pallas_call(
        _vq_kernel,
        grid=(nblk,),
        in_specs=[
            pl.BlockSpec((_BLK, _DIM), lambda i: (i, 0)),
            pl.BlockSpec((_DIM, _NE), lambda i: (0, 0)),
        ],
        out_specs=[
            pl.BlockSpec((_BLK, _DIM), lambda i: (i, 0)),
            pl.BlockSpec((_BLK, 1), lambda i: (i, 0)),
            pl.BlockSpec((1, 1), lambda i: (0, 0)),
        ],
        out_shape=[
            jax.ShapeDtypeStruct((n_tok, _DIM), jnp.float32),
            jax.ShapeDtypeStruct((n_tok, 1), jnp.int32),
            jax.ShapeDtypeStruct((1, 1), jnp.float32),
        ],
    )(flat, embed)
    quantize = q.reshape(input.shape)
    embed_ind = idx3.reshape(input.shape[:-1])
    diff_scalar = diff[0, 0] / jnp.float32(n_tok * _DIM)
    return (quantize, diff_scalar, embed_ind)
